# baseline (speedup 1.0000x reference)
import dataclasses
import functools

import jax
import jax.numpy as jnp
from jax import lax
from jax.experimental import pallas as pl
from jax.experimental.pallas import tpu as pltpu
from jax.experimental.pallas import tpu_sc as plsc

N = 10000
E = 320000
D = 128
NT = 2
K1 = 2
KEIG = 128
HID = 512
OUT = 64

NC = 2
NS = 16
NW = NC * NS
B = 128
NWIN = 80
EPW = B * NWIN
E_PAD = EPW * NW
CHUNKS = 4
CW = 64
ROWS_A = 640
ROWS_B = N - (NS - 1) * ROWS_A

_f32 = jnp.float32


def _a0_body(ev_ref, x_ref, o_ref):
    @pl.when(pl.program_id(0) == 0)
    def _():
        o_ref[...] = jnp.zeros_like(o_ref)
    o_ref[...] += lax.dot_general(
        ev_ref[...], x_ref[...], (((0,), (0,)), ((), ())),
        preferred_element_type=_f32)


def _a1_body(ev_ref, xs_ref, t_ref, evals_ref, o0, o1, o2, o3):
    coef = jnp.exp(-jnp.abs(t_ref[...])[:, None] * evals_ref[...][None, :])
    outs = (o0, o1, o2, o3)
    for ti in range(NT):
        xdt = jnp.dot(ev_ref[...], xs_ref[...] * coef[ti][:, None],
                      preferred_element_type=_f32)
        outs[2 * ti][...] = xdt[:, :CW]
        outs[2 * ti + 1][...] = xdt[:, CW:]


def _diffuse(x, evecs, t, evals):
    nb = 10
    bn = N // nb
    xs = pl.pallas_call(
        _a0_body,
        grid=(nb,),
        in_specs=[pl.BlockSpec((bn, KEIG), lambda i: (i, 0)),
                  pl.BlockSpec((bn, D), lambda i: (i, 0))],
        out_specs=pl.BlockSpec((KEIG, D), lambda i: (0, 0)),
        out_shape=jax.ShapeDtypeStruct((KEIG, D), _f32),
    )(evecs, x)
    xd_chunks = pl.pallas_call(
        _a1_body,
        grid=(nb,),
        in_specs=[pl.BlockSpec((bn, KEIG), lambda i: (i, 0)),
                  pl.BlockSpec((KEIG, D), lambda i: (0, 0)),
                  pl.BlockSpec((NT,), lambda i: (0,)),
                  pl.BlockSpec((KEIG,), lambda i: (0,))],
        out_specs=[pl.BlockSpec((bn, CW), lambda i: (i, 0))] * CHUNKS,
        out_shape=[jax.ShapeDtypeStruct((N, CW), _f32)] * CHUNKS,
    )(evecs, xs, t, evals)
    return xd_chunks


def _sc_compiler_params():
    cp = pltpu.CompilerParams()
    if "needs_layout_passes" in pltpu.CompilerParams.__dataclass_fields__:
        cp = dataclasses.replace(cp, needs_layout_passes=False)
    if "use_tc_tiling_on_sc" in pltpu.CompilerParams.__dataclass_fields__:
        cp = dataclasses.replace(cp, use_tc_tiling_on_sc=False)
    return cp


def _sc_mesh():
    return plsc.VectorSubcoreMesh(core_axis_name="c", subcore_axis_name="s",
                                  num_cores=NC, num_subcores=NS)


def _sc_conv_body(xd0, xd1, xd2, xd3, edata, onesb_h, zrows, zdeg,
                  p_out, deg_out,
                  ed_a, ed_b, onesb, rows_a, rows_b, s0_v, s1_v,
                  acc0, acc1, accd, sem_ea, sem_eb, sem_ga, sem_gb):
    cid = lax.axis_index("c")
    sid = lax.axis_index("s")
    wid = cid * NS + sid
    xd_refs = (xd0, xd1, xd2, xd3)

    pltpu.sync_copy(onesb_h, onesb)

    def start_ed(w, ed, sem):
        pltpu.async_copy(edata.at[wid, w], ed, sem)

    def wait_ed(ed, sem):
        pltpu.make_async_copy(edata.at[wid, 0], ed, sem).wait()

    def per_tile_rows(f):
        @pl.when(sid < NS - 1)
        def _():
            f(sid * ROWS_A, ROWS_A)

        @pl.when(sid == NS - 1)
        def _():
            f((NS - 1) * ROWS_A, ROWS_B)

    for chunk in range(CHUNKS):
        def zero_accs(off, size):
            pltpu.sync_copy(zrows.at[pl.ds(0, size)], acc0.at[pl.ds(off, size)])
            pltpu.sync_copy(zrows.at[pl.ds(0, size)], acc1.at[pl.ds(off, size)])
            if chunk == 0:
                pltpu.sync_copy(zdeg.at[pl.ds(0, size)],
                                accd.at[pl.ds(off, size)])
        per_tile_rows(zero_accs)
        plsc.subcore_barrier()

        xd_c = xd_refs[chunk]

        def start_gather(ed, rows, sem):
            pltpu.async_copy(xd_c.at[ed.at[0]], rows, sem)

        def wait_gather(rows, sem):
            pltpu.make_async_copy(xd_c.at[ed_a.at[0]], rows, sem).wait()

        def compute_and_scatter(ed, rows):
            @plsc.parallel_loop(0, B, unroll=8)
            def _(e):
                e_splat = jnp.full((16,), e, jnp.int32)
                w0s = plsc.bitcast(
                    plsc.load_gather(ed, [jnp.full((16,), 2, jnp.int32),
                                          e_splat]), _f32)
                w1s = plsc.bitcast(
                    plsc.load_gather(ed, [jnp.full((16,), 3, jnp.int32),
                                          e_splat]), _f32)
                for c in range(CW // 16):
                    seg = rows[e, pl.ds(c * 16, 16)]
                    s0_v[e, pl.ds(c * 16, 16)] = seg * w0s
                    s1_v[e, pl.ds(c * 16, 16)] = seg * w1s

            pltpu.sync_copy(s0_v, acc0.at[ed.at[1]], add=True)
            pltpu.sync_copy(s1_v, acc1.at[ed.at[1]], add=True)
            if chunk == 0:
                pltpu.sync_copy(onesb, accd.at[ed.at[1]], add=True)

        start_ed(0, ed_a, sem_ea)
        wait_ed(ed_a, sem_ea)
        start_gather(ed_a, rows_a, sem_ga)
        start_ed(1, ed_b, sem_eb)

        @pl.loop(0, NWIN, step=2)
        def _(w):
            wait_ed(ed_b, sem_eb)
            start_gather(ed_b, rows_b, sem_gb)
            wait_gather(rows_a, sem_ga)
            compute_and_scatter(ed_a, rows_a)

            @pl.when(w + 2 < NWIN)
            def _():
                start_ed(w + 2, ed_a, sem_ea)
            wait_gather(rows_b, sem_gb)
            compute_and_scatter(ed_b, rows_b)

            @pl.when(w + 2 < NWIN)
            def _():
                wait_ed(ed_a, sem_ea)
                start_gather(ed_a, rows_a, sem_ga)

            @pl.when(w + 3 < NWIN)
            def _():
                start_ed(w + 3, ed_b, sem_eb)

        plsc.subcore_barrier()

        def write_out(off, size):
            for j, acc in ((0, acc0), (1, acc1)):
                seg_base = ((cid * K1 + j) * CHUNKS + chunk) * N
                pltpu.sync_copy(acc.at[pl.ds(off, size)],
                                p_out.at[pl.ds(seg_base + off, size)])
            if chunk == 0:
                pltpu.sync_copy(accd.at[pl.ds(off, size)],
                                deg_out.at[pl.ds(cid * N + off, size)])
        per_tile_rows(write_out)
        plsc.subcore_barrier()


def _sc_conv(xd_chunks, edata, onesb_h, zrows, zdeg):
    kern = pl.kernel(
        _sc_conv_body,
        out_type=[jax.ShapeDtypeStruct((NC * K1 * CHUNKS * N, CW), _f32),
                  jax.ShapeDtypeStruct((NC * N, 1), _f32)],
        mesh=_sc_mesh(),
        scratch_types=[
            pltpu.VMEM((4, B), jnp.int32),
            pltpu.VMEM((4, B), jnp.int32),
            pltpu.VMEM((B, 1), _f32),
            pltpu.VMEM((B, CW), _f32),
            pltpu.VMEM((B, CW), _f32),
            pltpu.VMEM((B, CW), _f32),
            pltpu.VMEM((B, CW), _f32),
            pltpu.VMEM_SHARED((N, CW), _f32),
            pltpu.VMEM_SHARED((N, CW), _f32),
            pltpu.VMEM_SHARED((N, 1), _f32),
            pltpu.SemaphoreType.DMA,
            pltpu.SemaphoreType.DMA,
            pltpu.SemaphoreType.DMA,
            pltpu.SemaphoreType.DMA,
        ],
        compiler_params=_sc_compiler_params(),
    )
    return kern(*xd_chunks, edata, onesb_h, zrows, zdeg)


def _c_body(p_ref, deg0_ref, deg1_ref, degc_ref, w1_ref, b1_ref, w2_ref,
            b2_ref, o_ref):
    deg = jnp.maximum(deg0_ref[0, 0] + deg1_ref[0, 0] - degc_ref[0, 0], 1.0)
    parts = []
    for j in range(K1):
        for c in range(CHUNKS):
            parts.append(p_ref[(0 * K1 + j) * CHUNKS + c]
                         + p_ref[(1 * K1 + j) * CHUNKS + c])
    h = jnp.concatenate(parts, axis=1) / deg[:, None]
    z = jnp.maximum(h @ w1_ref[...] + b1_ref[...][None, :], 0.0)
    o = z @ w2_ref[...] + b2_ref[...][None, :]
    nrm = jnp.sqrt(jnp.sum(o * o, axis=1, keepdims=True))
    o_ref[...] = o / jnp.maximum(nrm, 1e-12)


def _mlp_head(p16, deg, degc, W1, b1, W2, b2):
    nb = 10
    bn = N // nb
    deg0 = deg[:N, 0].reshape(nb, 1, bn)
    deg1 = deg[N:, 0].reshape(nb, 1, bn)
    degc = degc.reshape(nb, 1, bn)
    return pl.pallas_call(
        _c_body,
        grid=(nb,),
        in_specs=[pl.BlockSpec((NC * K1 * CHUNKS, bn, CW), lambda i: (0, i, 0)),
                  pl.BlockSpec((1, 1, bn), lambda i: (i, 0, 0)),
                  pl.BlockSpec((1, 1, bn), lambda i: (i, 0, 0)),
                  pl.BlockSpec((1, 1, bn), lambda i: (i, 0, 0)),
                  pl.BlockSpec((K1 * NT * D, HID), lambda i: (0, 0)),
                  pl.BlockSpec((HID,), lambda i: (0,)),
                  pl.BlockSpec((HID, OUT), lambda i: (0, 0)),
                  pl.BlockSpec((OUT,), lambda i: (0,))],
        out_specs=pl.BlockSpec((bn, OUT), lambda i: (i, 0)),
        out_shape=jax.ShapeDtypeStruct((N, OUT), _f32),
    )(p16, deg0, deg1, degc, W1, b1, W2, b2)


def kernel(x, edge_index, evals, evecs, t, kernel_w, W1, b1, W2, b2):
    xd_chunks = _diffuse(x, evecs, t, evals)

    pad = E_PAD - E
    fill_idx = (jnp.arange(pad, dtype=jnp.int32) % N)
    srcp = jnp.concatenate([edge_index[0].astype(jnp.int32),
                            fill_idx]).reshape(NW, NWIN, B)
    dstp = jnp.concatenate([edge_index[1].astype(jnp.int32),
                            fill_idx]).reshape(NW, NWIN, B)
    zpad = jnp.zeros((pad,), _f32)
    w0b = lax.bitcast_convert_type(
        jnp.concatenate([kernel_w[0], zpad]), jnp.int32).reshape(NW, NWIN, B)
    w1b = lax.bitcast_convert_type(
        jnp.concatenate([kernel_w[1], zpad]), jnp.int32).reshape(NW, NWIN, B)
    edata = jnp.stack([srcp, dstp, w0b, w1b], axis=2)
    onesb_h = jnp.ones((B, 1), _f32)
    zrows = jnp.zeros((ROWS_A, CW), _f32)
    zdeg = jnp.zeros((ROWS_A, 1), _f32)
    degc = (jnp.arange(N) < pad).astype(_f32)

    p_out, deg = _sc_conv(xd_chunks, edata, onesb_h, zrows, zdeg)
    p16 = p_out.reshape(NC * K1 * CHUNKS, N, CW)
    return _mlp_head(p16, deg, degc, W1, b1, W2, b2)


# speedup vs baseline: 1.0289x; 1.0289x over previous
import dataclasses
import functools

import jax
import jax.numpy as jnp
from jax import lax
from jax.experimental import pallas as pl
from jax.experimental.pallas import tpu as pltpu
from jax.experimental.pallas import tpu_sc as plsc

N = 10000
E = 320000
D = 128
NT = 2
K1 = 2
KEIG = 128
HID = 512
OUT = 64

NC = 2
NS = 16
NW = NC * NS
B = 128
NWIN = 80
EPW = B * NWIN
E_PAD = EPW * NW
CHUNKS = 4
CW = 64
ROWS_A = 640
ROWS_B = N - (NS - 1) * ROWS_A

_f32 = jnp.float32


def _a0_body(ev_ref, x_ref, o_ref):
    @pl.when(pl.program_id(0) == 0)
    def _():
        o_ref[...] = jnp.zeros_like(o_ref)
    o_ref[...] += lax.dot_general(
        ev_ref[...], x_ref[...], (((0,), (0,)), ((), ())),
        preferred_element_type=_f32)


def _a1_body(ev_ref, xs_ref, t_ref, evals_ref, o0, o1, o2, o3):
    coef = jnp.exp(-jnp.abs(t_ref[...])[:, None] * evals_ref[...][None, :])
    outs = (o0, o1, o2, o3)
    for ti in range(NT):
        xdt = jnp.dot(ev_ref[...], xs_ref[...] * coef[ti][:, None],
                      preferred_element_type=_f32)
        outs[2 * ti][...] = xdt[:, :CW]
        outs[2 * ti + 1][...] = xdt[:, CW:]


def _diffuse(x, evecs, t, evals):
    nb = 10
    bn = N // nb
    xs = pl.pallas_call(
        _a0_body,
        grid=(nb,),
        in_specs=[pl.BlockSpec((bn, KEIG), lambda i: (i, 0)),
                  pl.BlockSpec((bn, D), lambda i: (i, 0))],
        out_specs=pl.BlockSpec((KEIG, D), lambda i: (0, 0)),
        out_shape=jax.ShapeDtypeStruct((KEIG, D), _f32),
    )(evecs, x)
    xd_chunks = pl.pallas_call(
        _a1_body,
        grid=(nb,),
        in_specs=[pl.BlockSpec((bn, KEIG), lambda i: (i, 0)),
                  pl.BlockSpec((KEIG, D), lambda i: (0, 0)),
                  pl.BlockSpec((NT,), lambda i: (0,)),
                  pl.BlockSpec((KEIG,), lambda i: (0,))],
        out_specs=[pl.BlockSpec((bn, CW), lambda i: (i, 0))] * CHUNKS,
        out_shape=[jax.ShapeDtypeStruct((N, CW), _f32)] * CHUNKS,
    )(evecs, xs, t, evals)
    return xd_chunks


def _sc_compiler_params():
    cp = pltpu.CompilerParams()
    if "needs_layout_passes" in pltpu.CompilerParams.__dataclass_fields__:
        cp = dataclasses.replace(cp, needs_layout_passes=False)
    if "use_tc_tiling_on_sc" in pltpu.CompilerParams.__dataclass_fields__:
        cp = dataclasses.replace(cp, use_tc_tiling_on_sc=False)
    return cp


def _sc_mesh():
    return plsc.VectorSubcoreMesh(core_axis_name="c", subcore_axis_name="s",
                                  num_cores=NC, num_subcores=NS)


def _sc_conv_body(xd0, xd1, xd2, xd3, edata, onesb_h, zrows, zdeg,
                  p_out, deg_out,
                  ed_a, ed_b, onesb, rows_a, rows_b, s0_v, s1_v,
                  acc0, acc1, accd, sem_ea, sem_eb, sem_ga, sem_gb, sem_s):
    cid = lax.axis_index("c")
    sid = lax.axis_index("s")
    wid = cid * NS + sid
    xd_refs = (xd0, xd1, xd2, xd3)

    pltpu.sync_copy(onesb_h, onesb)

    def start_ed(w, ed, sem):
        pltpu.async_copy(edata.at[wid, w], ed, sem)

    def wait_ed(ed, sem):
        pltpu.make_async_copy(edata.at[wid, 0], ed, sem).wait()

    def per_tile_rows(f):
        @pl.when(sid < NS - 1)
        def _():
            f(sid * ROWS_A, ROWS_A)

        @pl.when(sid == NS - 1)
        def _():
            f((NS - 1) * ROWS_A, ROWS_B)

    for chunk in range(CHUNKS):
        def zero_accs(off, size):
            pltpu.sync_copy(zrows.at[pl.ds(0, size)], acc0.at[pl.ds(off, size)])
            pltpu.sync_copy(zrows.at[pl.ds(0, size)], acc1.at[pl.ds(off, size)])
            if chunk == 0:
                pltpu.sync_copy(zdeg.at[pl.ds(0, size)],
                                accd.at[pl.ds(off, size)])
        per_tile_rows(zero_accs)
        plsc.subcore_barrier()

        xd_c = xd_refs[chunk]

        def start_gather(ed, rows, sem):
            pltpu.async_copy(xd_c.at[ed.at[0]], rows, sem)

        def wait_gather(rows, sem):
            pltpu.make_async_copy(xd_c.at[ed_a.at[0]], rows, sem).wait()

        def scale_pass(ed, rows, sbuf, wrow):
            @plsc.parallel_loop(0, B, unroll=4)
            def _(e):
                e_splat = jnp.full((16,), e, jnp.int32)
                ws = plsc.bitcast(
                    plsc.load_gather(ed, [jnp.full((16,), wrow, jnp.int32),
                                          e_splat]), _f32)
                for c in range(CW // 16):
                    sbuf[e, pl.ds(c * 16, 16)] = (rows[e, pl.ds(c * 16, 16)]
                                                  * ws)

        def compute_and_scatter(ed, rows):
            scale_pass(ed, rows, s0_v, 2)
            cp0 = pltpu.async_copy(s0_v, acc0.at[ed.at[1]], sem_s, add=True)
            if chunk == 0:
                cpd = pltpu.async_copy(onesb, accd.at[ed.at[1]], sem_s,
                                       add=True)
            scale_pass(ed, rows, s1_v, 3)
            cp0.wait()
            if chunk == 0:
                cpd.wait()
            pltpu.sync_copy(s1_v, acc1.at[ed.at[1]], add=True)

        start_ed(0, ed_a, sem_ea)
        wait_ed(ed_a, sem_ea)
        start_gather(ed_a, rows_a, sem_ga)
        start_ed(1, ed_b, sem_eb)

        @pl.loop(0, NWIN, step=2)
        def _(w):
            wait_ed(ed_b, sem_eb)
            start_gather(ed_b, rows_b, sem_gb)
            wait_gather(rows_a, sem_ga)
            compute_and_scatter(ed_a, rows_a)

            @pl.when(w + 2 < NWIN)
            def _():
                start_ed(w + 2, ed_a, sem_ea)
            wait_gather(rows_b, sem_gb)
            compute_and_scatter(ed_b, rows_b)

            @pl.when(w + 2 < NWIN)
            def _():
                wait_ed(ed_a, sem_ea)
                start_gather(ed_a, rows_a, sem_ga)

            @pl.when(w + 3 < NWIN)
            def _():
                start_ed(w + 3, ed_b, sem_eb)

        plsc.subcore_barrier()

        def write_out(off, size):
            for j, acc in ((0, acc0), (1, acc1)):
                seg_base = ((cid * K1 + j) * CHUNKS + chunk) * N
                pltpu.sync_copy(acc.at[pl.ds(off, size)],
                                p_out.at[pl.ds(seg_base + off, size)])
            if chunk == 0:
                pltpu.sync_copy(accd.at[pl.ds(off, size)],
                                deg_out.at[pl.ds(cid * N + off, size)])
        per_tile_rows(write_out)
        plsc.subcore_barrier()


def _sc_conv(xd_chunks, edata, onesb_h, zrows, zdeg):
    kern = pl.kernel(
        _sc_conv_body,
        out_type=[jax.ShapeDtypeStruct((NC * K1 * CHUNKS * N, CW), _f32),
                  jax.ShapeDtypeStruct((NC * N, 1), _f32)],
        mesh=_sc_mesh(),
        scratch_types=[
            pltpu.VMEM((4, B), jnp.int32),
            pltpu.VMEM((4, B), jnp.int32),
            pltpu.VMEM((B, 1), _f32),
            pltpu.VMEM((B, CW), _f32),
            pltpu.VMEM((B, CW), _f32),
            pltpu.VMEM((B, CW), _f32),
            pltpu.VMEM((B, CW), _f32),
            pltpu.VMEM_SHARED((N, CW), _f32),
            pltpu.VMEM_SHARED((N, CW), _f32),
            pltpu.VMEM_SHARED((N, 1), _f32),
            pltpu.SemaphoreType.DMA,
            pltpu.SemaphoreType.DMA,
            pltpu.SemaphoreType.DMA,
            pltpu.SemaphoreType.DMA,
            pltpu.SemaphoreType.DMA,
        ],
        compiler_params=_sc_compiler_params(),
    )
    return kern(*xd_chunks, edata, onesb_h, zrows, zdeg)


def _c_body(p_ref, deg0_ref, deg1_ref, degc_ref, w1_ref, b1_ref, w2_ref,
            b2_ref, o_ref):
    deg = jnp.maximum(deg0_ref[0, 0] + deg1_ref[0, 0] - degc_ref[0, 0], 1.0)
    parts = []
    for j in range(K1):
        for c in range(CHUNKS):
            parts.append(p_ref[(0 * K1 + j) * CHUNKS + c]
                         + p_ref[(1 * K1 + j) * CHUNKS + c])
    h = jnp.concatenate(parts, axis=1) / deg[:, None]
    z = jnp.maximum(h @ w1_ref[...] + b1_ref[...][None, :], 0.0)
    o = z @ w2_ref[...] + b2_ref[...][None, :]
    nrm = jnp.sqrt(jnp.sum(o * o, axis=1, keepdims=True))
    o_ref[...] = o / jnp.maximum(nrm, 1e-12)


def _mlp_head(p16, deg, degc, W1, b1, W2, b2):
    nb = 10
    bn = N // nb
    deg0 = deg[:N, 0].reshape(nb, 1, bn)
    deg1 = deg[N:, 0].reshape(nb, 1, bn)
    degc = degc.reshape(nb, 1, bn)
    return pl.pallas_call(
        _c_body,
        grid=(nb,),
        in_specs=[pl.BlockSpec((NC * K1 * CHUNKS, bn, CW), lambda i: (0, i, 0)),
                  pl.BlockSpec((1, 1, bn), lambda i: (i, 0, 0)),
                  pl.BlockSpec((1, 1, bn), lambda i: (i, 0, 0)),
                  pl.BlockSpec((1, 1, bn), lambda i: (i, 0, 0)),
                  pl.BlockSpec((K1 * NT * D, HID), lambda i: (0, 0)),
                  pl.BlockSpec((HID,), lambda i: (0,)),
                  pl.BlockSpec((HID, OUT), lambda i: (0, 0)),
                  pl.BlockSpec((OUT,), lambda i: (0,))],
        out_specs=pl.BlockSpec((bn, OUT), lambda i: (i, 0)),
        out_shape=jax.ShapeDtypeStruct((N, OUT), _f32),
    )(p16, deg0, deg1, degc, W1, b1, W2, b2)


def kernel(x, edge_index, evals, evecs, t, kernel_w, W1, b1, W2, b2):
    xd_chunks = _diffuse(x, evecs, t, evals)

    pad = E_PAD - E
    fill_idx = (jnp.arange(pad, dtype=jnp.int32) % N)
    srcp = jnp.concatenate([edge_index[0].astype(jnp.int32),
                            fill_idx]).reshape(NW, NWIN, B)
    dstp = jnp.concatenate([edge_index[1].astype(jnp.int32),
                            fill_idx]).reshape(NW, NWIN, B)
    zpad = jnp.zeros((pad,), _f32)
    w0b = lax.bitcast_convert_type(
        jnp.concatenate([kernel_w[0], zpad]), jnp.int32).reshape(NW, NWIN, B)
    w1b = lax.bitcast_convert_type(
        jnp.concatenate([kernel_w[1], zpad]), jnp.int32).reshape(NW, NWIN, B)
    edata = jnp.stack([srcp, dstp, w0b, w1b], axis=2)
    onesb_h = jnp.ones((B, 1), _f32)
    zrows = jnp.zeros((ROWS_A, CW), _f32)
    zdeg = jnp.zeros((ROWS_A, 1), _f32)
    degc = (jnp.arange(N) < pad).astype(_f32)

    p_out, deg = _sc_conv(xd_chunks, edata, onesb_h, zrows, zdeg)
    p16 = p_out.reshape(NC * K1 * CHUNKS, N, CW)
    return _mlp_head(p16, deg, degc, W1, b1, W2, b2)


# speedup vs baseline: 1.1363x; 1.1043x over previous
import dataclasses
import functools

import jax
import jax.numpy as jnp
from jax import lax
from jax.experimental import pallas as pl
from jax.experimental.pallas import tpu as pltpu
from jax.experimental.pallas import tpu_sc as plsc

N = 10000
E = 320000
D = 128
NT = 2
K1 = 2
KEIG = 128
HID = 512
OUT = 64

NC = 2
NS = 16
NW = NC * NS
B = 128
NWIN = 80
EPW = B * NWIN
E_PAD = EPW * NW
CHUNKS = 4
CW = 64
ROWS_A = 640
ROWS_B = N - (NS - 1) * ROWS_A

_f32 = jnp.float32


def _a0_body(ev_ref, x_ref, o_ref):
    @pl.when(pl.program_id(0) == 0)
    def _():
        o_ref[...] = jnp.zeros_like(o_ref)
    o_ref[...] += lax.dot_general(
        ev_ref[...], x_ref[...], (((0,), (0,)), ((), ())),
        preferred_element_type=_f32)


def _a1_body(ev_ref, xs_ref, t_ref, evals_ref, o0, o1, o2, o3):
    coef = jnp.exp(-jnp.abs(t_ref[...])[:, None] * evals_ref[...][None, :])
    outs = (o0, o1, o2, o3)
    for ti in range(NT):
        xdt = jnp.dot(ev_ref[...], xs_ref[...] * coef[ti][:, None],
                      preferred_element_type=_f32)
        outs[2 * ti][...] = xdt[:, :CW]
        outs[2 * ti + 1][...] = xdt[:, CW:]


def _diffuse(x, evecs, t, evals):
    nb = 10
    bn = N // nb
    xs = pl.pallas_call(
        _a0_body,
        grid=(nb,),
        in_specs=[pl.BlockSpec((bn, KEIG), lambda i: (i, 0)),
                  pl.BlockSpec((bn, D), lambda i: (i, 0))],
        out_specs=pl.BlockSpec((KEIG, D), lambda i: (0, 0)),
        out_shape=jax.ShapeDtypeStruct((KEIG, D), _f32),
    )(evecs, x)
    xd_chunks = pl.pallas_call(
        _a1_body,
        grid=(nb,),
        in_specs=[pl.BlockSpec((bn, KEIG), lambda i: (i, 0)),
                  pl.BlockSpec((KEIG, D), lambda i: (0, 0)),
                  pl.BlockSpec((NT,), lambda i: (0,)),
                  pl.BlockSpec((KEIG,), lambda i: (0,))],
        out_specs=[pl.BlockSpec((bn, CW), lambda i: (i, 0))] * CHUNKS,
        out_shape=[jax.ShapeDtypeStruct((N, CW), _f32)] * CHUNKS,
    )(evecs, xs, t, evals)
    return xd_chunks


def _sc_compiler_params():
    cp = pltpu.CompilerParams()
    if "needs_layout_passes" in pltpu.CompilerParams.__dataclass_fields__:
        cp = dataclasses.replace(cp, needs_layout_passes=False)
    if "use_tc_tiling_on_sc" in pltpu.CompilerParams.__dataclass_fields__:
        cp = dataclasses.replace(cp, use_tc_tiling_on_sc=False)
    return cp


def _sc_mesh():
    return plsc.VectorSubcoreMesh(core_axis_name="c", subcore_axis_name="s",
                                  num_cores=NC, num_subcores=NS)


def _sc_conv_body(xd0, xd1, xd2, xd3, edata, onesb_h, zrows, zdeg,
                  p_out, deg_out,
                  ed_a, ed_b, onesb, rows_a, rows_b, s01_v,
                  acc, accd, sem_ea, sem_eb, sem_ga, sem_gb, sem_s):
    cid = lax.axis_index("c")
    sid = lax.axis_index("s")
    wid = cid * NS + sid
    xd_refs = (xd0, xd1, xd2, xd3)

    pltpu.sync_copy(onesb_h, onesb)

    def start_ed(w, ed, sem):
        pltpu.async_copy(edata.at[wid, w], ed, sem)

    def wait_ed(ed, sem):
        pltpu.make_async_copy(edata.at[wid, 0], ed, sem).wait()

    def per_tile_rows(f):
        @pl.when(sid < NS - 1)
        def _():
            f(sid * ROWS_A, ROWS_A)

        @pl.when(sid == NS - 1)
        def _():
            f((NS - 1) * ROWS_A, ROWS_B)

    for chunk in range(CHUNKS):
        def zero_accs(off, size):
            pltpu.sync_copy(zrows.at[pl.ds(0, size)], acc.at[pl.ds(off, size)])
            if chunk == 0:
                pltpu.sync_copy(zdeg.at[pl.ds(0, size)],
                                accd.at[pl.ds(off, size)])
        per_tile_rows(zero_accs)
        plsc.subcore_barrier()

        xd_c = xd_refs[chunk]

        def start_gather(ed, rows, sem):
            pltpu.async_copy(xd_c.at[ed.at[0]], rows, sem)

        def wait_gather(rows, sem):
            pltpu.make_async_copy(xd_c.at[ed_a.at[0]], rows, sem).wait()

        def compute_and_scatter(ed, rows):
            if chunk == 0:
                cpd = pltpu.async_copy(onesb, accd.at[ed.at[1]], sem_s,
                                       add=True)

            @plsc.parallel_loop(0, B, unroll=4)
            def _(e):
                e_splat = jnp.full((16,), e, jnp.int32)
                w0s = plsc.bitcast(
                    plsc.load_gather(ed, [jnp.full((16,), 2, jnp.int32),
                                          e_splat]), _f32)
                w1s = plsc.bitcast(
                    plsc.load_gather(ed, [jnp.full((16,), 3, jnp.int32),
                                          e_splat]), _f32)
                for c in range(CW // 16):
                    seg = rows[e, pl.ds(c * 16, 16)]
                    s01_v[e, pl.ds(c * 16, 16)] = seg * w0s
                    s01_v[e, pl.ds(CW + c * 16, 16)] = seg * w1s

            if chunk == 0:
                cpd.wait()
            pltpu.sync_copy(s01_v, acc.at[ed.at[1]], add=True)

        start_ed(0, ed_a, sem_ea)
        wait_ed(ed_a, sem_ea)
        start_gather(ed_a, rows_a, sem_ga)
        start_ed(1, ed_b, sem_eb)

        @pl.loop(0, NWIN, step=2)
        def _(w):
            wait_ed(ed_b, sem_eb)
            start_gather(ed_b, rows_b, sem_gb)
            wait_gather(rows_a, sem_ga)
            compute_and_scatter(ed_a, rows_a)

            @pl.when(w + 2 < NWIN)
            def _():
                start_ed(w + 2, ed_a, sem_ea)
            wait_gather(rows_b, sem_gb)
            compute_and_scatter(ed_b, rows_b)

            @pl.when(w + 2 < NWIN)
            def _():
                wait_ed(ed_a, sem_ea)
                start_gather(ed_a, rows_a, sem_ga)

            @pl.when(w + 3 < NWIN)
            def _():
                start_ed(w + 3, ed_b, sem_eb)

        plsc.subcore_barrier()

        def write_out(off, size):
            seg_base = (cid * CHUNKS + chunk) * N
            pltpu.sync_copy(acc.at[pl.ds(off, size)],
                            p_out.at[pl.ds(seg_base + off, size)])
            if chunk == 0:
                pltpu.sync_copy(accd.at[pl.ds(off, size)],
                                deg_out.at[pl.ds(cid * N + off, size)])
        per_tile_rows(write_out)
        plsc.subcore_barrier()


def _sc_conv(xd_chunks, edata, onesb_h, zrows, zdeg):
    kern = pl.kernel(
        _sc_conv_body,
        out_type=[jax.ShapeDtypeStruct((NC * CHUNKS * N, K1 * CW), _f32),
                  jax.ShapeDtypeStruct((NC * N, 1), _f32)],
        mesh=_sc_mesh(),
        scratch_types=[
            pltpu.VMEM((4, B), jnp.int32),
            pltpu.VMEM((4, B), jnp.int32),
            pltpu.VMEM((B, 1), _f32),
            pltpu.VMEM((B, CW), _f32),
            pltpu.VMEM((B, CW), _f32),
            pltpu.VMEM((B, K1 * CW), _f32),
            pltpu.VMEM_SHARED((N, K1 * CW), _f32),
            pltpu.VMEM_SHARED((N, 1), _f32),
            pltpu.SemaphoreType.DMA,
            pltpu.SemaphoreType.DMA,
            pltpu.SemaphoreType.DMA,
            pltpu.SemaphoreType.DMA,
            pltpu.SemaphoreType.DMA,
        ],
        compiler_params=_sc_compiler_params(),
    )
    return kern(*xd_chunks, edata, onesb_h, zrows, zdeg)


def _c_body(p_ref, deg0_ref, deg1_ref, degc_ref, w1_ref, b1_ref, w2_ref,
            b2_ref, o_ref):
    deg = jnp.maximum(deg0_ref[0, 0] + deg1_ref[0, 0] - degc_ref[0, 0], 1.0)
    parts = []
    for j in range(K1):
        for c in range(CHUNKS):
            cols = pl.ds(j * CW, CW)
            parts.append(p_ref[0 * CHUNKS + c, :, cols]
                         + p_ref[1 * CHUNKS + c, :, cols])
    h = jnp.concatenate(parts, axis=1) / deg[:, None]
    z = jnp.maximum(h @ w1_ref[...] + b1_ref[...][None, :], 0.0)
    o = z @ w2_ref[...] + b2_ref[...][None, :]
    nrm = jnp.sqrt(jnp.sum(o * o, axis=1, keepdims=True))
    o_ref[...] = o / jnp.maximum(nrm, 1e-12)


def _mlp_head(p16, deg, degc, W1, b1, W2, b2):
    nb = 10
    bn = N // nb
    deg0 = deg[:N, 0].reshape(nb, 1, bn)
    deg1 = deg[N:, 0].reshape(nb, 1, bn)
    degc = degc.reshape(nb, 1, bn)
    return pl.pallas_call(
        _c_body,
        grid=(nb,),
        in_specs=[pl.BlockSpec((NC * CHUNKS, bn, K1 * CW), lambda i: (0, i, 0)),
                  pl.BlockSpec((1, 1, bn), lambda i: (i, 0, 0)),
                  pl.BlockSpec((1, 1, bn), lambda i: (i, 0, 0)),
                  pl.BlockSpec((1, 1, bn), lambda i: (i, 0, 0)),
                  pl.BlockSpec((K1 * NT * D, HID), lambda i: (0, 0)),
                  pl.BlockSpec((HID,), lambda i: (0,)),
                  pl.BlockSpec((HID, OUT), lambda i: (0, 0)),
                  pl.BlockSpec((OUT,), lambda i: (0,))],
        out_specs=pl.BlockSpec((bn, OUT), lambda i: (i, 0)),
        out_shape=jax.ShapeDtypeStruct((N, OUT), _f32),
    )(p16, deg0, deg1, degc, W1, b1, W2, b2)


def kernel(x, edge_index, evals, evecs, t, kernel_w, W1, b1, W2, b2):
    xd_chunks = _diffuse(x, evecs, t, evals)

    pad = E_PAD - E
    fill_idx = (jnp.arange(pad, dtype=jnp.int32) % N)
    srcp = jnp.concatenate([edge_index[0].astype(jnp.int32),
                            fill_idx]).reshape(NW, NWIN, B)
    dstp = jnp.concatenate([edge_index[1].astype(jnp.int32),
                            fill_idx]).reshape(NW, NWIN, B)
    zpad = jnp.zeros((pad,), _f32)
    w0b = lax.bitcast_convert_type(
        jnp.concatenate([kernel_w[0], zpad]), jnp.int32).reshape(NW, NWIN, B)
    w1b = lax.bitcast_convert_type(
        jnp.concatenate([kernel_w[1], zpad]), jnp.int32).reshape(NW, NWIN, B)
    edata = jnp.stack([srcp, dstp, w0b, w1b], axis=2)
    onesb_h = jnp.ones((B, 1), _f32)
    zrows = jnp.zeros((ROWS_A, K1 * CW), _f32)
    zdeg = jnp.zeros((ROWS_A, 1), _f32)
    degc = (jnp.arange(N) < pad).astype(_f32)

    p_out, deg = _sc_conv(xd_chunks, edata, onesb_h, zrows, zdeg)
    p16 = p_out.reshape(NC * CHUNKS, N, K1 * CW)
    return _mlp_head(p16, deg, degc, W1, b1, W2, b2)
